# baseline (speedup 1.0000x reference)
import functools

import jax
import jax.numpy as jnp
from jax import lax
from jax.experimental import pallas as pl
from jax.experimental.pallas import tpu as pltpu
from jax.experimental.pallas import tpu_sc as plsc

VOCAB = 100000
D = 128
BATCH = 4
SEQ = 2048
TOTAL = BATCH * SEQ
L = 16
VECS = D // L

_info = plsc.get_sparse_core_info()
NC = _info.num_cores
NS = _info.num_subcores
NW = NC * NS

S_PER_W = SEQ // NW


def _sc_body(x_hbm, table_hbm, pos_hbm, out_hbm,
             idx_v, pos_v, rows_v, sem_i, sem_p, sem_g, sem_o):
    wid = lax.axis_index("s") * NC + lax.axis_index("c")
    base_s = wid * S_PER_W

    idx_c = [
        pltpu.async_copy(x_hbm.at[b, pl.ds(base_s, S_PER_W)],
                         idx_v.at[b], sem_i)
        for b in range(BATCH)
    ]
    pos_c = pltpu.async_copy(pos_hbm.at[pl.ds(base_s, S_PER_W)], pos_v, sem_p)
    for c in idx_c:
        c.wait()

    g = [
        pltpu.async_copy(table_hbm.at[idx_v.at[b]], rows_v.at[b], sem_g)
        for b in range(BATCH)
    ]
    pos_c.wait()

    out = [None] * BATCH
    for b in range(BATCH):
        g[b].wait()

        def add_row(r, b=b):
            for c in range(VECS):
                sl = pl.ds(c * L, L)
                plsc.addupdate(rows_v.at[b, r, sl], pos_v[r, sl])

        pl.loop(0, S_PER_W, unroll=4)(add_row)
        out[b] = pltpu.async_copy(
            rows_v.at[b], out_hbm.at[pl.ds(b * SEQ + base_s, S_PER_W)], sem_o)

    for b in range(BATCH):
        out[b].wait()


@jax.jit
def _sc_call(x_flat, table, pos_emb):
    mesh = plsc.VectorSubcoreMesh(core_axis_name="c", subcore_axis_name="s")
    kfn = functools.partial(
        pl.kernel,
        mesh=mesh,
        out_type=jax.ShapeDtypeStruct((TOTAL, D), jnp.float32),
        scratch_types=[
            pltpu.VMEM((BATCH, S_PER_W), jnp.int32),
            pltpu.VMEM((S_PER_W, D), jnp.float32),
            pltpu.VMEM((BATCH, S_PER_W, D), jnp.float32),
            pltpu.SemaphoreType.DMA,
            pltpu.SemaphoreType.DMA,
            pltpu.SemaphoreType.DMA,
            pltpu.SemaphoreType.DMA,
        ],
    )(_sc_body)
    return kfn(x_flat, table, pos_emb)


def kernel(x, table, pos_emb):
    out = _sc_call(x.astype(jnp.int32), table, pos_emb)
    return out.reshape(BATCH, SEQ, D)


# speedup vs baseline: 1.0279x; 1.0279x over previous
import functools

import jax
import jax.numpy as jnp
from jax import lax
from jax.experimental import pallas as pl
from jax.experimental.pallas import tpu as pltpu
from jax.experimental.pallas import tpu_sc as plsc

VOCAB = 100000
D = 128
BATCH = 4
SEQ = 2048
TOTAL = BATCH * SEQ
L = 16
VECS = D // L

_info = plsc.get_sparse_core_info()
NC = _info.num_cores
NS = _info.num_subcores
NW = NC * NS

S_PER_W = SEQ // NW


def _sc_body(x_hbm, table_hbm, pos_hbm, out_hbm,
             idx_v, pos_v, rows_v, sem_i, sem_p, sem_g, sem_o):
    wid = lax.axis_index("s") * NC + lax.axis_index("c")
    base_s = wid * S_PER_W

    idx_c = [
        pltpu.async_copy(x_hbm.at[b, pl.ds(base_s, S_PER_W)],
                         idx_v.at[b], sem_i)
        for b in range(BATCH)
    ]
    pos_c = pltpu.async_copy(pos_hbm.at[pl.ds(base_s, S_PER_W)], pos_v, sem_p)
    for c in idx_c:
        c.wait()

    g = [
        pltpu.async_copy(table_hbm.at[idx_v.at[b]], rows_v.at[b], sem_g)
        for b in range(BATCH)
    ]
    pos_c.wait()

    out = [None] * BATCH
    for b in range(BATCH):
        g[b].wait()

        def add_row(r, b=b):
            for c in range(VECS):
                sl = pl.ds(c * L, L)
                plsc.addupdate(rows_v.at[b, r, sl], pos_v[r, sl])

        pl.loop(0, S_PER_W, unroll=2)(add_row)
        out[b] = pltpu.async_copy(
            rows_v.at[b], out_hbm.at[pl.ds(b * SEQ + base_s, S_PER_W)], sem_o)

    for b in range(BATCH):
        out[b].wait()


@jax.jit
def _sc_call(x_flat, table, pos_emb):
    mesh = plsc.VectorSubcoreMesh(core_axis_name="c", subcore_axis_name="s")
    kfn = functools.partial(
        pl.kernel,
        mesh=mesh,
        out_type=jax.ShapeDtypeStruct((TOTAL, D), jnp.float32),
        scratch_types=[
            pltpu.VMEM((BATCH, S_PER_W), jnp.int32),
            pltpu.VMEM((S_PER_W, D), jnp.float32),
            pltpu.VMEM((BATCH, S_PER_W, D), jnp.float32),
            pltpu.SemaphoreType.DMA,
            pltpu.SemaphoreType.DMA,
            pltpu.SemaphoreType.DMA,
            pltpu.SemaphoreType.DMA,
        ],
    )(_sc_body)
    return kfn(x_flat, table, pos_emb)


def kernel(x, table, pos_emb):
    out = _sc_call(x.astype(jnp.int32), table, pos_emb)
    return out.reshape(BATCH, SEQ, D)


# speedup vs baseline: 1.0413x; 1.0131x over previous
import functools

import jax
import jax.numpy as jnp
from jax import lax
from jax.experimental import pallas as pl
from jax.experimental.pallas import tpu as pltpu
from jax.experimental.pallas import tpu_sc as plsc

VOCAB = 100000
D = 128
BATCH = 4
SEQ = 2048
TOTAL = BATCH * SEQ
L = 16
VECS = D // L

_info = plsc.get_sparse_core_info()
NC = _info.num_cores
NS = _info.num_subcores
NW = NC * NS

S_PER_W = SEQ // NW


def _sc_body(x_hbm, table_hbm, pos_hbm, out_hbm,
             idx_v, pos_v, rows_v, sem_i, sem_p, sem_g, sem_o):
    wid = lax.axis_index("s") * NC + lax.axis_index("c")
    base_s = wid * S_PER_W

    idx_c = [
        pltpu.async_copy(x_hbm.at[b, pl.ds(base_s, S_PER_W)],
                         idx_v.at[b], sem_i)
        for b in range(BATCH)
    ]
    pos_c = pltpu.async_copy(pos_hbm.at[pl.ds(base_s, S_PER_W)], pos_v, sem_p)
    for c in idx_c:
        c.wait()

    g = [
        pltpu.async_copy(table_hbm.at[idx_v.at[b]], rows_v.at[b], sem_g)
        for b in range(BATCH)
    ]
    pos_c.wait()

    out = [None] * BATCH
    for b in range(BATCH):
        g[b].wait()

        def add_row(r, b=b):
            for c in range(VECS):
                sl = pl.ds(c * L, L)
                plsc.addupdate(rows_v.at[b, r, sl], pos_v[r, sl])

        pl.loop(0, S_PER_W)(add_row)
        out[b] = pltpu.async_copy(
            rows_v.at[b], out_hbm.at[pl.ds(b * SEQ + base_s, S_PER_W)], sem_o)

    for b in range(BATCH):
        out[b].wait()


@jax.jit
def _sc_call(x_flat, table, pos_emb):
    mesh = plsc.VectorSubcoreMesh(core_axis_name="c", subcore_axis_name="s")
    kfn = functools.partial(
        pl.kernel,
        mesh=mesh,
        out_type=jax.ShapeDtypeStruct((TOTAL, D), jnp.float32),
        scratch_types=[
            pltpu.VMEM((BATCH, S_PER_W), jnp.int32),
            pltpu.VMEM((S_PER_W, D), jnp.float32),
            pltpu.VMEM((BATCH, S_PER_W, D), jnp.float32),
            pltpu.SemaphoreType.DMA,
            pltpu.SemaphoreType.DMA,
            pltpu.SemaphoreType.DMA,
            pltpu.SemaphoreType.DMA,
        ],
    )(_sc_body)
    return kfn(x_flat, table, pos_emb)


def kernel(x, table, pos_emb):
    out = _sc_call(x.astype(jnp.int32), table, pos_emb)
    return out.reshape(BATCH, SEQ, D)
